# baseline (speedup 1.0000x reference)
import functools

import jax
import jax.numpy as jnp
from jax.experimental import pallas as pl
from jax.experimental.pallas import tpu as pltpu


def _gru_kernel(x_ref, xn_ref, emb_ref, k_ref, rec_ref, badd_ref, b1h_ref,
                wd_ref, bd_ref, out_ref, table_ref, mx_ref, ohn_ref,
                hs_ref, h_ref, *, chunk, b, u, v):
    i = pl.program_id(0)
    p = jax.lax.rem(i, 2)

    @pl.when(i == 0)
    def _init():
        table_ref[...] = (
            jnp.dot(emb_ref[...], k_ref[...],
                    preferred_element_type=jnp.float32) + badd_ref[...])
        h_ref[...] = jnp.zeros_like(h_ref)
        xb = x_ref[...]
        iota = jax.lax.broadcasted_iota(jnp.int32, (chunk, b, v), 2)
        oh = (xb[:, :, None] == iota).astype(jnp.float32).reshape(chunk * b, v)
        mx_ref[0] = jnp.dot(oh, table_ref[...],
                            preferred_element_type=jnp.float32).astype(
                                jnp.bfloat16)

    xn = xn_ref[...]
    iota2 = jax.lax.broadcasted_iota(jnp.int32, (chunk, b, v), 2)
    ohn_ref[...] = (xn[:, :, None] == iota2).astype(jnp.float32).reshape(
        chunk * b, v)
    h = h_ref[...]
    rec_z = rec_ref[:, :u]
    rec_r = rec_ref[:, u:2 * u]
    rec_h = rec_ref[:, 2 * u:]
    b1h = b1h_ref[...]
    table = table_ref[...]
    for t in range(chunk):
        mx = mx_ref[p, pl.ds(t * b, b), :].astype(jnp.float32)
        mi_r = jnp.dot(h, rec_r, preferred_element_type=jnp.float32)
        mi_h = jnp.dot(h, rec_h, preferred_element_type=jnp.float32)
        mi_z = jnp.dot(h, rec_z, preferred_element_type=jnp.float32)
        r = jax.nn.sigmoid(mx[:, u:2 * u] + mi_r)
        z = jax.nn.sigmoid(mx[:, :u] + mi_z)
        hh = jnp.tanh(mx[:, 2 * u:] + r * (mi_h + b1h))
        h = hh + z * (h - hh)
        hs_ref[pl.ds(t * b, b), :] = h
        if t % 8 == 7:
            om = jnp.dot(hs_ref[pl.ds((t - 7) * b, 8 * b), :], wd_ref[...],
                         preferred_element_type=jnp.float32) + bd_ref[...]
            out_ref[:, t - 7:t + 1, :] = jnp.swapaxes(
                om.reshape(8, b, v), 0, 1)
        mx_ref[1 - p, pl.ds(t * b, b), :] = jnp.dot(
            ohn_ref[pl.ds(t * b, b), :], table,
            preferred_element_type=jnp.float32).astype(jnp.bfloat16)
    h_ref[...] = h


def kernel(x, emb, kernel, rec_kernel, bias, Wd, bd):
    b, s = x.shape
    v, e = emb.shape
    u = rec_kernel.shape[0]
    chunk = 32
    nchunks = s // chunk

    xT = jnp.swapaxes(x, 0, 1).astype(jnp.int32)
    badd = bias[0:1] + jnp.concatenate(
        [bias[1:2, :2 * u], jnp.zeros((1, u), bias.dtype)], axis=1)
    b1h = bias[1:2, 2 * u:]
    bd2 = bd.reshape(1, v)

    out = pl.pallas_call(
        functools.partial(_gru_kernel, chunk=chunk, b=b, u=u, v=v),
        grid=(nchunks,),
        in_specs=[
            pl.BlockSpec((chunk, b), lambda i: (i, 0)),
            pl.BlockSpec((chunk, b),
                         lambda i: (jnp.minimum(i + 1, nchunks - 1), 0)),
            pl.BlockSpec((v, e), lambda i: (0, 0)),
            pl.BlockSpec((e, 3 * u), lambda i: (0, 0)),
            pl.BlockSpec((u, 3 * u), lambda i: (0, 0)),
            pl.BlockSpec((1, 3 * u), lambda i: (0, 0)),
            pl.BlockSpec((1, u), lambda i: (0, 0)),
            pl.BlockSpec((u, v), lambda i: (0, 0)),
            pl.BlockSpec((1, v), lambda i: (0, 0)),
        ],
        out_specs=pl.BlockSpec((b, chunk, v), lambda i: (0, i, 0)),
        out_shape=jax.ShapeDtypeStruct((b, s, v), jnp.float32),
        scratch_shapes=[
            pltpu.VMEM((v, 3 * u), jnp.float32),
            pltpu.VMEM((2, chunk * b, 3 * u), jnp.bfloat16),
            pltpu.VMEM((chunk * b, v), jnp.float32),
            pltpu.VMEM((chunk * b, u), jnp.float32),
            pltpu.VMEM((b, u), jnp.float32),
        ],
        compiler_params=pltpu.CompilerParams(
            dimension_semantics=("arbitrary",)),
    )(xT, xT, emb, kernel, rec_kernel, badd, b1h, Wd, bd2)

    return out


# speedup vs baseline: 1.0028x; 1.0028x over previous
import functools

import jax
import jax.numpy as jnp
from jax.experimental import pallas as pl
from jax.experimental.pallas import tpu as pltpu


def _gru_kernel(x_ref, xn_ref, emb_ref, k_ref, rec_ref, badd_ref, b1h_ref,
                wd_ref, bd_ref, out_ref, table_ref, mx_ref, ohn_ref,
                h_ref, *, chunk, b, u, v):
    i = pl.program_id(0)
    p = jax.lax.rem(i, 2)

    @pl.when(i == 0)
    def _init():
        table_ref[...] = (
            jnp.dot(emb_ref[...], k_ref[...],
                    preferred_element_type=jnp.float32) + badd_ref[...])
        h_ref[...] = jnp.zeros_like(h_ref)
        xb = x_ref[...]
        iota = jax.lax.broadcasted_iota(jnp.int32, (chunk, b, v), 2)
        oh = (xb[:, :, None] == iota).astype(jnp.float32).reshape(chunk * b, v)
        mx_ref[0] = jnp.dot(oh, table_ref[...],
                            preferred_element_type=jnp.float32).astype(
                                jnp.bfloat16)

    xn = xn_ref[...]
    iota2 = jax.lax.broadcasted_iota(jnp.int32, (chunk, b, v), 2)
    ohn_ref[...] = (xn[:, :, None] == iota2).astype(jnp.float32).reshape(
        chunk * b, v)
    h = h_ref[...]
    rec_zr = rec_ref[:, :2 * u]
    rec_h = rec_ref[:, 2 * u:]
    b1h = b1h_ref[...]
    table = table_ref[...]
    hbuf = []
    for t in range(chunk):
        mx = mx_ref[p, pl.ds(t * b, b), :].astype(jnp.float32)
        mi_zr = jnp.dot(h, rec_zr, preferred_element_type=jnp.float32)
        mi_h = jnp.dot(h, rec_h, preferred_element_type=jnp.float32)
        zr = jax.nn.sigmoid(mx[:, :2 * u] + mi_zr)
        z = zr[:, :u]
        r = zr[:, u:]
        hh = jnp.tanh(mx[:, 2 * u:] + r * (mi_h + b1h))
        h = hh + z * (h - hh)
        hbuf.append(h)
        if t % 8 == 7:
            om = jnp.dot(jnp.concatenate(hbuf, axis=0), wd_ref[...],
                         preferred_element_type=jnp.float32) + bd_ref[...]
            out_ref[:, t - 7:t + 1, :] = jnp.swapaxes(
                om.reshape(8, b, v), 0, 1)
            hbuf = []
        mx_ref[1 - p, pl.ds(t * b, b), :] = jnp.dot(
            ohn_ref[pl.ds(t * b, b), :], table,
            preferred_element_type=jnp.float32).astype(jnp.bfloat16)
    h_ref[...] = h


def kernel(x, emb, kernel, rec_kernel, bias, Wd, bd):
    b, s = x.shape
    v, e = emb.shape
    u = rec_kernel.shape[0]
    chunk = 32
    nchunks = s // chunk

    xT = jnp.swapaxes(x, 0, 1).astype(jnp.int32)
    badd = bias[0:1] + jnp.concatenate(
        [bias[1:2, :2 * u], jnp.zeros((1, u), bias.dtype)], axis=1)
    b1h = bias[1:2, 2 * u:]
    bd2 = bd.reshape(1, v)

    out = pl.pallas_call(
        functools.partial(_gru_kernel, chunk=chunk, b=b, u=u, v=v),
        grid=(nchunks,),
        in_specs=[
            pl.BlockSpec((chunk, b), lambda i: (i, 0)),
            pl.BlockSpec((chunk, b),
                         lambda i: (jnp.minimum(i + 1, nchunks - 1), 0)),
            pl.BlockSpec((v, e), lambda i: (0, 0)),
            pl.BlockSpec((e, 3 * u), lambda i: (0, 0)),
            pl.BlockSpec((u, 3 * u), lambda i: (0, 0)),
            pl.BlockSpec((1, 3 * u), lambda i: (0, 0)),
            pl.BlockSpec((1, u), lambda i: (0, 0)),
            pl.BlockSpec((u, v), lambda i: (0, 0)),
            pl.BlockSpec((1, v), lambda i: (0, 0)),
        ],
        out_specs=pl.BlockSpec((b, chunk, v), lambda i: (0, i, 0)),
        out_shape=jax.ShapeDtypeStruct((b, s, v), jnp.float32),
        scratch_shapes=[
            pltpu.VMEM((v, 3 * u), jnp.float32),
            pltpu.VMEM((2, chunk * b, 3 * u), jnp.bfloat16),
            pltpu.VMEM((chunk * b, v), jnp.float32),
            pltpu.VMEM((b, u), jnp.float32),
        ],
        compiler_params=pltpu.CompilerParams(
            dimension_semantics=("arbitrary",)),
    )(xT, xT, emb, kernel, rec_kernel, badd, b1h, Wd, bd2)

    return out
